# baseline (speedup 1.0000x reference)
import functools

import jax
import jax.numpy as jnp
from jax import lax
from jax.experimental import pallas as pl
from jax.experimental.pallas import tpu as pltpu
from jax.experimental.pallas import tpu_sc as plsc

N = 10000
E = 320000
D = 128
B = 64
H = 128
C = 16
EPS = 1e-5

NC = 2
NS = 16
NW = NC * NS
EDGES_PER_TILE = E // NW
CH = 40
NCHUNK = EDGES_PER_TILE // CH
ROWS_A = 624
ROWS_LAST = N - 15 * ROWS_A
LAST_BASE = 15 * ROWS_A


def _edge_agg(kd, qv, src, dst, zer):
  mesh = plsc.VectorSubcoreMesh(
      core_axis_name="c", subcore_axis_name="s", num_cores=NC,
      num_subcores=NS)

  @functools.partial(
      pl.kernel,
      out_type=jax.ShapeDtypeStruct((NC, N, D), jnp.float32),
      mesh=mesh,
      scratch_types=[
          pltpu.VMEM((CH,), jnp.int32),
          pltpu.VMEM((CH,), jnp.int32),
          pltpu.VMEM((CH,), jnp.int32),
          pltpu.VMEM((CH,), jnp.int32),
          pltpu.VMEM((CH, D), jnp.float32),
          pltpu.VMEM((CH, D), jnp.float32),
          pltpu.VMEM((CH, 2 * D), jnp.float32),
          pltpu.VMEM((CH, 2 * D), jnp.float32),
          pltpu.VMEM((CH, D), jnp.float32),
          pltpu.VMEM_SHARED((N, D), jnp.float32),
          pltpu.SemaphoreType.DMA,
          pltpu.SemaphoreType.DMA,
          pltpu.SemaphoreType.DMA,
          pltpu.SemaphoreType.DMA,
          pltpu.SemaphoreType.DMA,
          pltpu.SemaphoreType.DMA,
      ],
  )
  def body(kd_hbm, qv_hbm, src_hbm, dst_hbm, zer_hbm, out_hbm,
           srcv0, srcv1, dstv0, dstv1, krows0, krows1, qvrows0, qvrows1,
           msg, agg, sem_i0, sem_i1, sem_k0, sem_qv0, sem_k1, sem_qv1):
    c = lax.axis_index("c")
    s = lax.axis_index("s")

    rbase = pl.multiple_of(s * ROWS_A, 8)

    @pl.when(s < NS - 1)
    def _():
      pltpu.sync_copy(zer_hbm.at[pl.ds(rbase, ROWS_A)],
                      agg.at[pl.ds(rbase, ROWS_A)])

    @pl.when(s == NS - 1)
    def _():
      pltpu.sync_copy(zer_hbm.at[pl.ds(LAST_BASE, ROWS_LAST)],
                      agg.at[pl.ds(LAST_BASE, ROWS_LAST)])

    plsc.subcore_barrier()

    wid = c * NS + s
    ebase = wid * EDGES_PER_TILE

    def issue_idx(g, srcv, dstv, sem_i):
      base = pl.multiple_of(ebase + g * CH, 8)
      pltpu.async_copy(src_hbm.at[pl.ds(base, CH)], srcv, sem_i)
      pltpu.async_copy(dst_hbm.at[pl.ds(base, CH)], dstv, sem_i)

    def wait_idx(srcv, dstv, sem_i):
      pltpu.make_async_copy(src_hbm.at[pl.ds(0, CH)], srcv, sem_i).wait()
      pltpu.make_async_copy(src_hbm.at[pl.ds(0, CH)], dstv, sem_i).wait()

    def issue_gather(srcv, dstv, krows, qvrows, sem_k, sem_qv):
      pltpu.async_copy(kd_hbm.at[dstv], krows, sem_k)
      pltpu.async_copy(qv_hbm.at[srcv], qvrows, sem_qv)

    def process(dstv, krows, qvrows, sem_k, sem_qv):
      pltpu.make_async_copy(kd_hbm.at[pl.ds(0, CH)], krows, sem_k).wait()
      pltpu.make_async_copy(qv_hbm.at[pl.ds(0, CH)], qvrows, sem_qv).wait()

      @plsc.parallel_loop(0, CH, step=1, unroll=8)
      def _(i):
        for j in range(D // 16):
          kk = krows[i, pl.ds(j * 16, 16)]
          qq = qvrows[i, pl.ds(j * 16, 16)]
          vv = qvrows[i, pl.ds(D + j * 16, 16)]
          msg[i, pl.ds(j * 16, 16)] = vv / (1.0 + jnp.exp(-(kk + qq)))

      pltpu.sync_copy(msg, agg.at[dstv], add=True)

    issue_idx(0, srcv0, dstv0, sem_i0)
    wait_idx(srcv0, dstv0, sem_i0)
    issue_gather(srcv0, dstv0, krows0, qvrows0, sem_k0, sem_qv0)
    issue_idx(1, srcv1, dstv1, sem_i1)

    def pair(it, carry):
      g = it * 2
      wait_idx(srcv1, dstv1, sem_i1)
      issue_gather(srcv1, dstv1, krows1, qvrows1, sem_k1, sem_qv1)
      process(dstv0, krows0, qvrows0, sem_k0, sem_qv0)

      @pl.when(g + 2 < NCHUNK)
      def _():
        issue_idx(g + 2, srcv0, dstv0, sem_i0)
        wait_idx(srcv0, dstv0, sem_i0)
        issue_gather(srcv0, dstv0, krows0, qvrows0, sem_k0, sem_qv0)

      process(dstv1, krows1, qvrows1, sem_k1, sem_qv1)

      @pl.when(g + 3 < NCHUNK)
      def _():
        issue_idx(g + 3, srcv1, dstv1, sem_i1)

      return carry

    lax.fori_loop(0, NCHUNK // 2, pair, 0)

    plsc.subcore_barrier()

    @pl.when(s < NS - 1)
    def _():
      pltpu.sync_copy(agg.at[pl.ds(rbase, ROWS_A)],
                      out_hbm.at[c, pl.ds(rbase, ROWS_A)])

    @pl.when(s == NS - 1)
    def _():
      pltpu.sync_copy(agg.at[pl.ds(LAST_BASE, ROWS_LAST)],
                      out_hbm.at[c, pl.ds(LAST_BASE, ROWS_LAST)])

  return body(kd, qv, src, dst, zer)


def _proj(h, wcat, bcat):
  def body(h_ref, w_ref, b_ref, kd_ref, qv_ref, xs_ref):
    cat = jnp.dot(h_ref[...], w_ref[...],
                  preferred_element_type=jnp.float32) + b_ref[...]
    kd_ref[...] = cat[:, :D]
    qv_ref[...] = cat[:, D:3 * D]
    xs_ref[...] = cat[:, 3 * D:]

  return pl.pallas_call(
      body,
      out_shape=[
          jax.ShapeDtypeStruct((N, D), jnp.float32),
          jax.ShapeDtypeStruct((N, 2 * D), jnp.float32),
          jax.ShapeDtypeStruct((N, D), jnp.float32),
      ],
  )(h, wcat, bcat)


def _relu_bn(parts, xs, g, b):
  def body(p_ref, xs_ref, g_ref, b_ref, hn_ref):
    h = jnp.maximum(p_ref[0] + p_ref[1] + xs_ref[...], 0.0)
    m = jnp.mean(h, axis=0, keepdims=True)
    v = jnp.mean((h - m) ** 2, axis=0, keepdims=True)
    hn_ref[...] = (h - m) * lax.rsqrt(v + EPS) * g_ref[...] + b_ref[...]

  return pl.pallas_call(
      body,
      out_shape=jax.ShapeDtypeStruct((N, D), jnp.float32),
  )(parts, xs, g, b)


def _head(hn_in, batch_r, batch_c, gap_g, gap_b, gsp_g, gsp_b,
          h1_w, h1_b, hn1_g, hn1_b, last_w, last_b):
  NEG = float(jnp.finfo(jnp.float32).min)

  def bn(t, gg, bb):
    m = jnp.mean(t, axis=0, keepdims=True)
    v = jnp.mean((t - m) ** 2, axis=0, keepdims=True)
    return (t - m) * lax.rsqrt(v + EPS) * gg + bb

  def body(hn_ref, br_ref, bc_ref,
           gapg_ref, gapb_ref, gspg_ref, gspb_ref,
           h1w_ref, h1b_ref, hn1g_ref, hn1b_ref, lw_ref, lb_ref, o_ref):
    hn = hn_ref[...]

    rows = lax.broadcasted_iota(jnp.int32, (B, N), 0)
    onehot = (rows == br_ref[...]).astype(jnp.float32)
    cnt = jnp.sum(onehot, axis=1, keepdims=True)
    gap = jnp.dot(onehot, hn, preferred_element_type=jnp.float32)
    gap = gap / jnp.maximum(cnt, 1.0)

    bcol = bc_ref[...]
    rows64 = lax.broadcasted_iota(jnp.int32, (B, 1), 0)

    def gsp_step(i, acc):
      m = jnp.max(jnp.where(bcol == i, hn, NEG), axis=0, keepdims=True)
      return jnp.where(rows64 == i, m, acc)

    gsp = lax.fori_loop(0, B, gsp_step, jnp.full((B, D), NEG, jnp.float32))
    gsp = jnp.where(cnt > 0.0, gsp, 0.0)

    gap = bn(gap, gapg_ref[...], gapb_ref[...])
    gsp = bn(gsp, gspg_ref[...], gspb_ref[...])
    out = jnp.concatenate([gap, gsp], axis=1)
    out = jnp.dot(out, h1w_ref[...],
                  preferred_element_type=jnp.float32) + h1b_ref[...]
    out = jnp.maximum(out, 0.0)
    out = bn(out, hn1g_ref[...], hn1b_ref[...])
    o_ref[...] = jnp.dot(out, lw_ref[...],
                         preferred_element_type=jnp.float32) + lb_ref[...]

  return pl.pallas_call(
      body,
      out_shape=jax.ShapeDtypeStruct((B, C), jnp.float32),
  )(hn_in, batch_r, batch_c, gap_g, gap_b, gsp_g, gsp_b,
    h1_w, h1_b, hn1_g, hn1_b, last_w, last_b)


def kernel(x, edge_index, batch, Wk1, Wq1, Wv1, Ws1, bk1, bq1, bv1, cb1,
           bn1_g, bn1_b, Wk2, Wq2, Wv2, Ws2, bk2, bq2, bv2, cb2, bn2_g,
           bn2_b, gap_g, gap_b, gsp_g, gsp_b, h1_W, h1_b, hn1_g, hn1_b,
           last_W, last_b):
  src = edge_index[0]
  dst = edge_index[1]
  zer = jnp.zeros((N, D), jnp.float32)

  w1 = jnp.concatenate([Wk1, Wq1, Wv1, Ws1], axis=1)
  b1 = jnp.concatenate([bk1, bq1, bv1, cb1]).reshape(1, 4 * D)
  w2 = jnp.concatenate([Wk2, Wq2, Wv2, Ws2], axis=1)
  b2 = jnp.concatenate([bk2, bq2, bv2, cb2]).reshape(1, 4 * D)

  kd1, qv1, xs1 = _proj(x, w1, b1)
  parts1 = _edge_agg(kd1, qv1, src, dst, zer)
  h1 = _relu_bn(parts1, xs1, bn1_g.reshape(1, D), bn1_b.reshape(1, D))
  kd2, qv2, xs2 = _proj(h1, w2, b2)
  parts2 = _edge_agg(kd2, qv2, src, dst, zer)
  h2 = _relu_bn(parts2, xs2, bn2_g.reshape(1, D), bn2_b.reshape(1, D))
  return _head(h2, batch.reshape(1, N), batch.reshape(N, 1),
               gap_g.reshape(1, D), gap_b.reshape(1, D),
               gsp_g.reshape(1, D), gsp_b.reshape(1, D),
               h1_W, h1_b.reshape(1, H), hn1_g.reshape(1, H),
               hn1_b.reshape(1, H), last_W, last_b.reshape(1, C))


# speedup vs baseline: 1.1914x; 1.1914x over previous
import functools

import jax
import jax.numpy as jnp
from jax import lax
from jax.experimental import pallas as pl
from jax.experimental.pallas import tpu as pltpu
from jax.experimental.pallas import tpu_sc as plsc

N = 10000
E = 320000
D = 128
B = 64
H = 128
C = 16
EPS = 1e-5

NC = 2
NS = 16
NW = NC * NS
EDGES_PER_TILE = E // NW
CH = 40
NCHUNK = EDGES_PER_TILE // CH
ROWS_A = 624
ROWS_LAST = N - 15 * ROWS_A
LAST_BASE = 15 * ROWS_A


def _edge_agg(kd, qv, src, dst, zer):
  mesh = plsc.VectorSubcoreMesh(
      core_axis_name="c", subcore_axis_name="s", num_cores=NC,
      num_subcores=NS)

  @functools.partial(
      pl.kernel,
      out_type=jax.ShapeDtypeStruct((NC, N, D), jnp.float32),
      mesh=mesh,
      scratch_types=[
          pltpu.VMEM((CH,), jnp.int32),
          pltpu.VMEM((CH,), jnp.int32),
          pltpu.VMEM((CH,), jnp.int32),
          pltpu.VMEM((CH,), jnp.int32),
          pltpu.VMEM((CH, D), jnp.float32),
          pltpu.VMEM((CH, D), jnp.float32),
          pltpu.VMEM((CH, 2 * D), jnp.float32),
          pltpu.VMEM((CH, 2 * D), jnp.float32),
          pltpu.VMEM((CH, D), jnp.float32),
          pltpu.VMEM_SHARED((N, D), jnp.float32),
          pltpu.SemaphoreType.DMA,
          pltpu.SemaphoreType.DMA,
          pltpu.SemaphoreType.DMA,
          pltpu.SemaphoreType.DMA,
          pltpu.SemaphoreType.DMA,
          pltpu.SemaphoreType.DMA,
      ],
  )
  def body(kd_hbm, qv_hbm, src_hbm, dst_hbm, zer_hbm, out_hbm,
           srcv0, srcv1, dstv0, dstv1, krows0, krows1, qvrows0, qvrows1,
           msg, agg, sem_i0, sem_i1, sem_k0, sem_qv0, sem_k1, sem_qv1):
    c = lax.axis_index("c")
    s = lax.axis_index("s")

    rbase = pl.multiple_of(s * ROWS_A, 8)

    @pl.when(s < NS - 1)
    def _():
      pltpu.sync_copy(zer_hbm.at[pl.ds(rbase, ROWS_A)],
                      agg.at[pl.ds(rbase, ROWS_A)])

    @pl.when(s == NS - 1)
    def _():
      pltpu.sync_copy(zer_hbm.at[pl.ds(LAST_BASE, ROWS_LAST)],
                      agg.at[pl.ds(LAST_BASE, ROWS_LAST)])

    plsc.subcore_barrier()

    wid = c * NS + s
    ebase = wid * EDGES_PER_TILE

    def issue_idx(g, srcv, dstv, sem_i):
      base = pl.multiple_of(ebase + g * CH, 8)
      pltpu.async_copy(src_hbm.at[pl.ds(base, CH)], srcv, sem_i)
      pltpu.async_copy(dst_hbm.at[pl.ds(base, CH)], dstv, sem_i)

    def wait_idx(srcv, dstv, sem_i):
      pltpu.make_async_copy(src_hbm.at[pl.ds(0, CH)], srcv, sem_i).wait()
      pltpu.make_async_copy(src_hbm.at[pl.ds(0, CH)], dstv, sem_i).wait()

    def issue_gather(srcv, dstv, krows, qvrows, sem_k, sem_qv):
      pltpu.async_copy(kd_hbm.at[dstv], krows, sem_k)
      pltpu.async_copy(qv_hbm.at[srcv], qvrows, sem_qv)

    def process(dstv, krows, qvrows, sem_k, sem_qv):
      pltpu.make_async_copy(kd_hbm.at[pl.ds(0, CH)], krows, sem_k).wait()
      pltpu.make_async_copy(qv_hbm.at[pl.ds(0, CH)], qvrows, sem_qv).wait()

      @plsc.parallel_loop(0, CH, step=1, unroll=4)
      def _(i):
        for j in range(D // 16):
          kk = krows[i, pl.ds(j * 16, 16)]
          qq = qvrows[i, pl.ds(j * 16, 16)]
          vv = qvrows[i, pl.ds(D + j * 16, 16)]
          msg[i, pl.ds(j * 16, 16)] = vv / (1.0 + jnp.exp(-(kk + qq)))

      pltpu.sync_copy(msg, agg.at[dstv], add=True)

    issue_idx(0, srcv0, dstv0, sem_i0)
    wait_idx(srcv0, dstv0, sem_i0)
    issue_gather(srcv0, dstv0, krows0, qvrows0, sem_k0, sem_qv0)
    issue_idx(1, srcv1, dstv1, sem_i1)

    def pair(it, carry):
      g = it * 2
      wait_idx(srcv1, dstv1, sem_i1)
      issue_gather(srcv1, dstv1, krows1, qvrows1, sem_k1, sem_qv1)
      process(dstv0, krows0, qvrows0, sem_k0, sem_qv0)

      @pl.when(g + 2 < NCHUNK)
      def _():
        issue_idx(g + 2, srcv0, dstv0, sem_i0)
        wait_idx(srcv0, dstv0, sem_i0)
        issue_gather(srcv0, dstv0, krows0, qvrows0, sem_k0, sem_qv0)

      process(dstv1, krows1, qvrows1, sem_k1, sem_qv1)

      @pl.when(g + 3 < NCHUNK)
      def _():
        issue_idx(g + 3, srcv1, dstv1, sem_i1)

      return carry

    lax.fori_loop(0, NCHUNK // 2, pair, 0)

    plsc.subcore_barrier()

    @pl.when(s < NS - 1)
    def _():
      pltpu.sync_copy(agg.at[pl.ds(rbase, ROWS_A)],
                      out_hbm.at[c, pl.ds(rbase, ROWS_A)])

    @pl.when(s == NS - 1)
    def _():
      pltpu.sync_copy(agg.at[pl.ds(LAST_BASE, ROWS_LAST)],
                      out_hbm.at[c, pl.ds(LAST_BASE, ROWS_LAST)])

  return body(kd, qv, src, dst, zer)


def _proj(h, wcat, bcat):
  def body(h_ref, w_ref, b_ref, kd_ref, qv_ref, xs_ref):
    cat = jnp.dot(h_ref[...], w_ref[...],
                  preferred_element_type=jnp.float32) + b_ref[...]
    kd_ref[...] = cat[:, :D]
    qv_ref[...] = cat[:, D:3 * D]
    xs_ref[...] = cat[:, 3 * D:]

  return pl.pallas_call(
      body,
      out_shape=[
          jax.ShapeDtypeStruct((N, D), jnp.float32),
          jax.ShapeDtypeStruct((N, 2 * D), jnp.float32),
          jax.ShapeDtypeStruct((N, D), jnp.float32),
      ],
  )(h, wcat, bcat)


def _relu_bn(parts, xs, g, b):
  def body(p_ref, xs_ref, g_ref, b_ref, hn_ref):
    h = jnp.maximum(p_ref[0] + p_ref[1] + xs_ref[...], 0.0)
    m = jnp.mean(h, axis=0, keepdims=True)
    v = jnp.mean((h - m) ** 2, axis=0, keepdims=True)
    hn_ref[...] = (h - m) * lax.rsqrt(v + EPS) * g_ref[...] + b_ref[...]

  return pl.pallas_call(
      body,
      out_shape=jax.ShapeDtypeStruct((N, D), jnp.float32),
  )(parts, xs, g, b)


def _head(hn_in, batch_r, batch_c, gap_g, gap_b, gsp_g, gsp_b,
          h1_w, h1_b, hn1_g, hn1_b, last_w, last_b):
  NEG = float(jnp.finfo(jnp.float32).min)

  def bn(t, gg, bb):
    m = jnp.mean(t, axis=0, keepdims=True)
    v = jnp.mean((t - m) ** 2, axis=0, keepdims=True)
    return (t - m) * lax.rsqrt(v + EPS) * gg + bb

  def body(hn_ref, br_ref, bc_ref,
           gapg_ref, gapb_ref, gspg_ref, gspb_ref,
           h1w_ref, h1b_ref, hn1g_ref, hn1b_ref, lw_ref, lb_ref, o_ref):
    hn = hn_ref[...]

    rows = lax.broadcasted_iota(jnp.int32, (B, N), 0)
    onehot = (rows == br_ref[...]).astype(jnp.float32)
    cnt = jnp.sum(onehot, axis=1, keepdims=True)
    gap = jnp.dot(onehot, hn, preferred_element_type=jnp.float32)
    gap = gap / jnp.maximum(cnt, 1.0)

    bcol = bc_ref[...]
    rows64 = lax.broadcasted_iota(jnp.int32, (B, 1), 0)

    def gsp_step(i, acc):
      m = jnp.max(jnp.where(bcol == i, hn, NEG), axis=0, keepdims=True)
      return jnp.where(rows64 == i, m, acc)

    gsp = lax.fori_loop(0, B, gsp_step, jnp.full((B, D), NEG, jnp.float32))
    gsp = jnp.where(cnt > 0.0, gsp, 0.0)

    gap = bn(gap, gapg_ref[...], gapb_ref[...])
    gsp = bn(gsp, gspg_ref[...], gspb_ref[...])
    out = jnp.concatenate([gap, gsp], axis=1)
    out = jnp.dot(out, h1w_ref[...],
                  preferred_element_type=jnp.float32) + h1b_ref[...]
    out = jnp.maximum(out, 0.0)
    out = bn(out, hn1g_ref[...], hn1b_ref[...])
    o_ref[...] = jnp.dot(out, lw_ref[...],
                         preferred_element_type=jnp.float32) + lb_ref[...]

  return pl.pallas_call(
      body,
      out_shape=jax.ShapeDtypeStruct((B, C), jnp.float32),
  )(hn_in, batch_r, batch_c, gap_g, gap_b, gsp_g, gsp_b,
    h1_w, h1_b, hn1_g, hn1_b, last_w, last_b)


def kernel(x, edge_index, batch, Wk1, Wq1, Wv1, Ws1, bk1, bq1, bv1, cb1,
           bn1_g, bn1_b, Wk2, Wq2, Wv2, Ws2, bk2, bq2, bv2, cb2, bn2_g,
           bn2_b, gap_g, gap_b, gsp_g, gsp_b, h1_W, h1_b, hn1_g, hn1_b,
           last_W, last_b):
  src = edge_index[0]
  dst = edge_index[1]
  zer = jnp.zeros((N, D), jnp.float32)

  w1 = jnp.concatenate([Wk1, Wq1, Wv1, Ws1], axis=1)
  b1 = jnp.concatenate([bk1, bq1, bv1, cb1]).reshape(1, 4 * D)
  w2 = jnp.concatenate([Wk2, Wq2, Wv2, Ws2], axis=1)
  b2 = jnp.concatenate([bk2, bq2, bv2, cb2]).reshape(1, 4 * D)

  kd1, qv1, xs1 = _proj(x, w1, b1)
  parts1 = _edge_agg(kd1, qv1, src, dst, zer)
  h1 = _relu_bn(parts1, xs1, bn1_g.reshape(1, D), bn1_b.reshape(1, D))
  kd2, qv2, xs2 = _proj(h1, w2, b2)
  parts2 = _edge_agg(kd2, qv2, src, dst, zer)
  h2 = _relu_bn(parts2, xs2, bn2_g.reshape(1, D), bn2_b.reshape(1, D))
  return _head(h2, batch.reshape(1, N), batch.reshape(N, 1),
               gap_g.reshape(1, D), gap_b.reshape(1, D),
               gsp_g.reshape(1, D), gsp_b.reshape(1, D),
               h1_W, h1_b.reshape(1, H), hn1_g.reshape(1, H),
               hn1_b.reshape(1, H), last_W, last_b.reshape(1, C))


# speedup vs baseline: 1.2207x; 1.0246x over previous
import functools

import jax
import jax.numpy as jnp
from jax import lax
from jax.experimental import pallas as pl
from jax.experimental.pallas import tpu as pltpu
from jax.experimental.pallas import tpu_sc as plsc

N = 10000
E = 320000
D = 128
B = 64
H = 128
C = 16
EPS = 1e-5

NC = 2
NS = 16
NW = NC * NS
EDGES_PER_TILE = E // NW
CH = 40
NCHUNK = EDGES_PER_TILE // CH
ROWS_A = 624
ROWS_LAST = N - 15 * ROWS_A
LAST_BASE = 15 * ROWS_A


def _edge_agg(kd, qv, src, dst, zer):
  mesh = plsc.VectorSubcoreMesh(
      core_axis_name="c", subcore_axis_name="s", num_cores=NC,
      num_subcores=NS)

  @functools.partial(
      pl.kernel,
      out_type=jax.ShapeDtypeStruct((NC, N, D), jnp.float32),
      mesh=mesh,
      scratch_types=[
          pltpu.VMEM((EDGES_PER_TILE,), jnp.int32),
          pltpu.VMEM((CH,), jnp.int32),
          pltpu.VMEM((CH,), jnp.int32),
          pltpu.VMEM((CH, D), jnp.float32),
          pltpu.VMEM((CH, D), jnp.float32),
          pltpu.VMEM((CH, 2 * D), jnp.float32),
          pltpu.VMEM((CH, 2 * D), jnp.float32),
          pltpu.VMEM((CH, D), jnp.float32),
          pltpu.VMEM_SHARED((N, D), jnp.float32),
          pltpu.SemaphoreType.DMA,
          pltpu.SemaphoreType.DMA,
          pltpu.SemaphoreType.DMA,
          pltpu.SemaphoreType.DMA,
          pltpu.SemaphoreType.DMA,
          pltpu.SemaphoreType.DMA,
      ],
  )
  def body(kd_hbm, qv_hbm, src_hbm, dst_hbm, zer_hbm, out_hbm,
           srcv_all, dstv0, dstv1, krows0, krows1, qvrows0, qvrows1,
           msg, agg, sem_i0, sem_i1, sem_k0, sem_qv0, sem_k1, sem_qv1):
    c = lax.axis_index("c")
    s = lax.axis_index("s")

    rbase = pl.multiple_of(s * ROWS_A, 8)

    @pl.when(s < NS - 1)
    def _():
      pltpu.sync_copy(zer_hbm.at[pl.ds(rbase, ROWS_A)],
                      agg.at[pl.ds(rbase, ROWS_A)])

    @pl.when(s == NS - 1)
    def _():
      pltpu.sync_copy(zer_hbm.at[pl.ds(LAST_BASE, ROWS_LAST)],
                      agg.at[pl.ds(LAST_BASE, ROWS_LAST)])

    wid = c * NS + s
    ebase = pl.multiple_of(wid * EDGES_PER_TILE, 8)
    pltpu.sync_copy(src_hbm.at[pl.ds(ebase, EDGES_PER_TILE)], srcv_all)
    plsc.subcore_barrier()

    def issue_idx(g, dstv, sem_i):
      base = pl.multiple_of(ebase + g * CH, 8)
      pltpu.async_copy(dst_hbm.at[pl.ds(base, CH)], dstv, sem_i)

    def wait_idx(dstv, sem_i):
      pltpu.make_async_copy(dst_hbm.at[pl.ds(0, CH)], dstv, sem_i).wait()

    def issue_gather(g, dstv, krows, qvrows, sem_k, sem_qv):
      off = pl.multiple_of(g * CH, 8)
      pltpu.async_copy(kd_hbm.at[dstv], krows, sem_k)
      pltpu.async_copy(qv_hbm.at[srcv_all.at[pl.ds(off, CH)]], qvrows,
                       sem_qv)

    def process(dstv, krows, qvrows, sem_k, sem_qv):
      pltpu.make_async_copy(kd_hbm.at[pl.ds(0, CH)], krows, sem_k).wait()
      pltpu.make_async_copy(qv_hbm.at[pl.ds(0, CH)], qvrows, sem_qv).wait()

      @plsc.parallel_loop(0, CH, step=1, unroll=4)
      def _(i):
        for j in range(D // 16):
          kk = krows[i, pl.ds(j * 16, 16)]
          qq = qvrows[i, pl.ds(j * 16, 16)]
          vv = qvrows[i, pl.ds(D + j * 16, 16)]
          msg[i, pl.ds(j * 16, 16)] = vv / (1.0 + jnp.exp(-(kk + qq)))

      pltpu.sync_copy(msg, agg.at[dstv], add=True)

    issue_idx(0, dstv0, sem_i0)
    wait_idx(dstv0, sem_i0)
    issue_gather(0, dstv0, krows0, qvrows0, sem_k0, sem_qv0)
    issue_idx(1, dstv1, sem_i1)

    def pair(it, carry):
      g = it * 2
      wait_idx(dstv1, sem_i1)
      issue_gather(g + 1, dstv1, krows1, qvrows1, sem_k1, sem_qv1)
      process(dstv0, krows0, qvrows0, sem_k0, sem_qv0)

      @pl.when(g + 2 < NCHUNK)
      def _():
        issue_idx(g + 2, dstv0, sem_i0)
        wait_idx(dstv0, sem_i0)
        issue_gather(g + 2, dstv0, krows0, qvrows0, sem_k0, sem_qv0)

      process(dstv1, krows1, qvrows1, sem_k1, sem_qv1)

      @pl.when(g + 3 < NCHUNK)
      def _():
        issue_idx(g + 3, dstv1, sem_i1)

      return carry

    lax.fori_loop(0, NCHUNK // 2, pair, 0)

    plsc.subcore_barrier()

    @pl.when(s < NS - 1)
    def _():
      pltpu.sync_copy(agg.at[pl.ds(rbase, ROWS_A)],
                      out_hbm.at[c, pl.ds(rbase, ROWS_A)])

    @pl.when(s == NS - 1)
    def _():
      pltpu.sync_copy(agg.at[pl.ds(LAST_BASE, ROWS_LAST)],
                      out_hbm.at[c, pl.ds(LAST_BASE, ROWS_LAST)])

  return body(kd, qv, src, dst, zer)


def _proj(h, wcat, bcat):
  def body(h_ref, w_ref, b_ref, kd_ref, qv_ref, xs_ref):
    cat = jnp.dot(h_ref[...], w_ref[...],
                  preferred_element_type=jnp.float32) + b_ref[...]
    kd_ref[...] = cat[:, :D]
    qv_ref[...] = cat[:, D:3 * D]
    xs_ref[...] = cat[:, 3 * D:]

  return pl.pallas_call(
      body,
      out_shape=[
          jax.ShapeDtypeStruct((N, D), jnp.float32),
          jax.ShapeDtypeStruct((N, 2 * D), jnp.float32),
          jax.ShapeDtypeStruct((N, D), jnp.float32),
      ],
  )(h, wcat, bcat)


def _relu_bn(parts, xs, g, b):
  def body(p_ref, xs_ref, g_ref, b_ref, hn_ref):
    h = jnp.maximum(p_ref[0] + p_ref[1] + xs_ref[...], 0.0)
    m = jnp.mean(h, axis=0, keepdims=True)
    v = jnp.mean((h - m) ** 2, axis=0, keepdims=True)
    hn_ref[...] = (h - m) * lax.rsqrt(v + EPS) * g_ref[...] + b_ref[...]

  return pl.pallas_call(
      body,
      out_shape=jax.ShapeDtypeStruct((N, D), jnp.float32),
  )(parts, xs, g, b)


def _head(hn_in, batch_r, batch_c, gap_g, gap_b, gsp_g, gsp_b,
          h1_w, h1_b, hn1_g, hn1_b, last_w, last_b):
  NEG = float(jnp.finfo(jnp.float32).min)

  def bn(t, gg, bb):
    m = jnp.mean(t, axis=0, keepdims=True)
    v = jnp.mean((t - m) ** 2, axis=0, keepdims=True)
    return (t - m) * lax.rsqrt(v + EPS) * gg + bb

  def body(hn_ref, br_ref, bc_ref,
           gapg_ref, gapb_ref, gspg_ref, gspb_ref,
           h1w_ref, h1b_ref, hn1g_ref, hn1b_ref, lw_ref, lb_ref, o_ref):
    hn = hn_ref[...]

    rows = lax.broadcasted_iota(jnp.int32, (B, N), 0)
    onehot = (rows == br_ref[...]).astype(jnp.float32)
    cnt = jnp.sum(onehot, axis=1, keepdims=True)
    gap = jnp.dot(onehot, hn, preferred_element_type=jnp.float32)
    gap = gap / jnp.maximum(cnt, 1.0)

    bcol = bc_ref[...]
    rows64 = lax.broadcasted_iota(jnp.int32, (B, 1), 0)

    def gsp_step(i, acc):
      m = jnp.max(jnp.where(bcol == i, hn, NEG), axis=0, keepdims=True)
      return jnp.where(rows64 == i, m, acc)

    gsp = lax.fori_loop(0, B, gsp_step, jnp.full((B, D), NEG, jnp.float32))
    gsp = jnp.where(cnt > 0.0, gsp, 0.0)

    gap = bn(gap, gapg_ref[...], gapb_ref[...])
    gsp = bn(gsp, gspg_ref[...], gspb_ref[...])
    out = jnp.concatenate([gap, gsp], axis=1)
    out = jnp.dot(out, h1w_ref[...],
                  preferred_element_type=jnp.float32) + h1b_ref[...]
    out = jnp.maximum(out, 0.0)
    out = bn(out, hn1g_ref[...], hn1b_ref[...])
    o_ref[...] = jnp.dot(out, lw_ref[...],
                         preferred_element_type=jnp.float32) + lb_ref[...]

  return pl.pallas_call(
      body,
      out_shape=jax.ShapeDtypeStruct((B, C), jnp.float32),
  )(hn_in, batch_r, batch_c, gap_g, gap_b, gsp_g, gsp_b,
    h1_w, h1_b, hn1_g, hn1_b, last_w, last_b)


def kernel(x, edge_index, batch, Wk1, Wq1, Wv1, Ws1, bk1, bq1, bv1, cb1,
           bn1_g, bn1_b, Wk2, Wq2, Wv2, Ws2, bk2, bq2, bv2, cb2, bn2_g,
           bn2_b, gap_g, gap_b, gsp_g, gsp_b, h1_W, h1_b, hn1_g, hn1_b,
           last_W, last_b):
  src = edge_index[0]
  dst = edge_index[1]
  zer = jnp.zeros((N, D), jnp.float32)

  w1 = jnp.concatenate([Wk1, Wq1, Wv1, Ws1], axis=1)
  b1 = jnp.concatenate([bk1, bq1, bv1, cb1]).reshape(1, 4 * D)
  w2 = jnp.concatenate([Wk2, Wq2, Wv2, Ws2], axis=1)
  b2 = jnp.concatenate([bk2, bq2, bv2, cb2]).reshape(1, 4 * D)

  kd1, qv1, xs1 = _proj(x, w1, b1)
  parts1 = _edge_agg(kd1, qv1, src, dst, zer)
  h1 = _relu_bn(parts1, xs1, bn1_g.reshape(1, D), bn1_b.reshape(1, D))
  kd2, qv2, xs2 = _proj(h1, w2, b2)
  parts2 = _edge_agg(kd2, qv2, src, dst, zer)
  h2 = _relu_bn(parts2, xs2, bn2_g.reshape(1, D), bn2_b.reshape(1, D))
  return _head(h2, batch.reshape(1, N), batch.reshape(N, 1),
               gap_g.reshape(1, D), gap_b.reshape(1, D),
               gsp_g.reshape(1, D), gsp_b.reshape(1, D),
               h1_W, h1_b.reshape(1, H), hn1_g.reshape(1, H),
               hn1_b.reshape(1, H), last_W, last_b.reshape(1, C))


# speedup vs baseline: 1.2987x; 1.0639x over previous
import functools

import jax
import jax.numpy as jnp
from jax import lax
from jax.experimental import pallas as pl
from jax.experimental.pallas import tpu as pltpu
from jax.experimental.pallas import tpu_sc as plsc

N = 10000
E = 320000
D = 128
B = 64
H = 128
C = 16
EPS = 1e-5

NC = 2
NS = 16
NW = NC * NS
EDGES_PER_TILE = E // NW
CH = 40
NCHUNK = EDGES_PER_TILE // CH
ROWS_A = 624
ROWS_LAST = N - 15 * ROWS_A
LAST_BASE = 15 * ROWS_A


def _edge_agg(kd, qv, src, dst, zer):
  mesh = plsc.VectorSubcoreMesh(
      core_axis_name="c", subcore_axis_name="s", num_cores=NC,
      num_subcores=NS)

  @functools.partial(
      pl.kernel,
      out_type=jax.ShapeDtypeStruct((NC, N, D), jnp.float32),
      mesh=mesh,
      scratch_types=[
          pltpu.VMEM((EDGES_PER_TILE,), jnp.int32),
          pltpu.VMEM((CH,), jnp.int32),
          pltpu.VMEM((CH,), jnp.int32),
          pltpu.VMEM((CH,), jnp.int32),
          pltpu.VMEM((CH,), jnp.int32),
          pltpu.VMEM((CH, D), jnp.float32),
          pltpu.VMEM((CH, D), jnp.float32),
          pltpu.VMEM((CH, 2 * D), jnp.float32),
          pltpu.VMEM((CH, 2 * D), jnp.float32),
          pltpu.VMEM_SHARED((N, D), jnp.float32),
          pltpu.SemaphoreType.DMA,
          pltpu.SemaphoreType.DMA,
          pltpu.SemaphoreType.DMA,
          pltpu.SemaphoreType.DMA,
          pltpu.SemaphoreType.DMA,
          pltpu.SemaphoreType.DMA,
          pltpu.SemaphoreType.DMA,
          pltpu.SemaphoreType.DMA,
      ],
  )
  def body(kd_hbm, qv_hbm, src_hbm, dst_hbm, zer_hbm, out_hbm,
           srcv_all, dstv0, dstv1, dsc0, dsc1, krows0, krows1,
           qvrows0, qvrows1,
           agg, sem_i0, sem_i1, sem_k0, sem_qv0, sem_k1, sem_qv1,
           sem_s0, sem_s1):
    c = lax.axis_index("c")
    s = lax.axis_index("s")

    rbase = pl.multiple_of(s * ROWS_A, 8)

    @pl.when(s < NS - 1)
    def _():
      pltpu.sync_copy(zer_hbm.at[pl.ds(rbase, ROWS_A)],
                      agg.at[pl.ds(rbase, ROWS_A)])

    @pl.when(s == NS - 1)
    def _():
      pltpu.sync_copy(zer_hbm.at[pl.ds(LAST_BASE, ROWS_LAST)],
                      agg.at[pl.ds(LAST_BASE, ROWS_LAST)])

    wid = c * NS + s
    ebase = pl.multiple_of(wid * EDGES_PER_TILE, 8)
    pltpu.sync_copy(src_hbm.at[pl.ds(ebase, EDGES_PER_TILE)], srcv_all)
    plsc.subcore_barrier()

    def issue_idx(g, dstv, sem_i):
      base = pl.multiple_of(ebase + g * CH, 8)
      pltpu.async_copy(dst_hbm.at[pl.ds(base, CH)], dstv, sem_i)

    def wait_idx(dstv, sem_i):
      pltpu.make_async_copy(dst_hbm.at[pl.ds(0, CH)], dstv, sem_i).wait()

    def issue_gather(g, dstv, krows, qvrows, sem_k, sem_qv):
      off = pl.multiple_of(g * CH, 8)
      pltpu.async_copy(kd_hbm.at[dstv], krows, sem_k)
      pltpu.async_copy(qv_hbm.at[srcv_all.at[pl.ds(off, CH)]], qvrows,
                       sem_qv)

    def wait_scatter(krows, sem_s):
      pltpu.make_async_copy(krows, agg.at[pl.ds(0, CH)], sem_s).wait()

    def process(dstv, dsc, krows, qvrows, sem_k, sem_qv, sem_s):
      pltpu.make_async_copy(kd_hbm.at[pl.ds(0, CH)], krows, sem_k).wait()
      pltpu.make_async_copy(qv_hbm.at[pl.ds(0, CH)], qvrows, sem_qv).wait()
      for j in range(CH // 16):
        dsc[pl.ds(j * 16, 16)] = dstv[pl.ds(j * 16, 16)]
      if CH % 16:
        dsc[pl.ds(CH - 16, 16)] = dstv[pl.ds(CH - 16, 16)]

      @plsc.parallel_loop(0, CH, step=1, unroll=4)
      def _(i):
        for j in range(D // 16):
          kk = krows[i, pl.ds(j * 16, 16)]
          qq = qvrows[i, pl.ds(j * 16, 16)]
          vv = qvrows[i, pl.ds(D + j * 16, 16)]
          krows[i, pl.ds(j * 16, 16)] = vv / (1.0 + jnp.exp(-(kk + qq)))

      pltpu.async_copy(krows, agg.at[dsc], sem_s, add=True)

    issue_idx(0, dstv0, sem_i0)
    wait_idx(dstv0, sem_i0)
    issue_gather(0, dstv0, krows0, qvrows0, sem_k0, sem_qv0)
    issue_idx(1, dstv1, sem_i1)

    def pair(it, carry):
      g = it * 2
      @pl.when(g > 0)
      def _():
        wait_scatter(krows1, sem_s1)

      wait_idx(dstv1, sem_i1)
      issue_gather(g + 1, dstv1, krows1, qvrows1, sem_k1, sem_qv1)
      process(dstv0, dsc0, krows0, qvrows0, sem_k0, sem_qv0, sem_s0)

      @pl.when(g + 2 < NCHUNK)
      def _():
        wait_scatter(krows0, sem_s0)
        issue_idx(g + 2, dstv0, sem_i0)
        wait_idx(dstv0, sem_i0)
        issue_gather(g + 2, dstv0, krows0, qvrows0, sem_k0, sem_qv0)

      process(dstv1, dsc1, krows1, qvrows1, sem_k1, sem_qv1, sem_s1)

      @pl.when(g + 3 < NCHUNK)
      def _():
        issue_idx(g + 3, dstv1, sem_i1)

      return carry

    lax.fori_loop(0, NCHUNK // 2, pair, 0)

    wait_scatter(krows0, sem_s0)
    wait_scatter(krows1, sem_s1)
    plsc.subcore_barrier()

    @pl.when(s < NS - 1)
    def _():
      pltpu.sync_copy(agg.at[pl.ds(rbase, ROWS_A)],
                      out_hbm.at[c, pl.ds(rbase, ROWS_A)])

    @pl.when(s == NS - 1)
    def _():
      pltpu.sync_copy(agg.at[pl.ds(LAST_BASE, ROWS_LAST)],
                      out_hbm.at[c, pl.ds(LAST_BASE, ROWS_LAST)])

  return body(kd, qv, src, dst, zer)


def _proj(h, wcat, bcat):
  def body(h_ref, w_ref, b_ref, kd_ref, qv_ref, xs_ref):
    cat = jnp.dot(h_ref[...], w_ref[...],
                  preferred_element_type=jnp.float32) + b_ref[...]
    kd_ref[...] = cat[:, :D]
    qv_ref[...] = cat[:, D:3 * D]
    xs_ref[...] = cat[:, 3 * D:]

  return pl.pallas_call(
      body,
      out_shape=[
          jax.ShapeDtypeStruct((N, D), jnp.float32),
          jax.ShapeDtypeStruct((N, 2 * D), jnp.float32),
          jax.ShapeDtypeStruct((N, D), jnp.float32),
      ],
  )(h, wcat, bcat)


def _relu_bn(parts, xs, g, b):
  def body(p_ref, xs_ref, g_ref, b_ref, hn_ref):
    h = jnp.maximum(p_ref[0] + p_ref[1] + xs_ref[...], 0.0)
    m = jnp.mean(h, axis=0, keepdims=True)
    v = jnp.mean((h - m) ** 2, axis=0, keepdims=True)
    hn_ref[...] = (h - m) * lax.rsqrt(v + EPS) * g_ref[...] + b_ref[...]

  return pl.pallas_call(
      body,
      out_shape=jax.ShapeDtypeStruct((N, D), jnp.float32),
  )(parts, xs, g, b)


def _head(hn_in, batch_r, batch_c, gap_g, gap_b, gsp_g, gsp_b,
          h1_w, h1_b, hn1_g, hn1_b, last_w, last_b):
  NEG = float(jnp.finfo(jnp.float32).min)

  def bn(t, gg, bb):
    m = jnp.mean(t, axis=0, keepdims=True)
    v = jnp.mean((t - m) ** 2, axis=0, keepdims=True)
    return (t - m) * lax.rsqrt(v + EPS) * gg + bb

  def body(hn_ref, br_ref, bc_ref,
           gapg_ref, gapb_ref, gspg_ref, gspb_ref,
           h1w_ref, h1b_ref, hn1g_ref, hn1b_ref, lw_ref, lb_ref, o_ref):
    hn = hn_ref[...]

    rows = lax.broadcasted_iota(jnp.int32, (B, N), 0)
    onehot = (rows == br_ref[...]).astype(jnp.float32)
    cnt = jnp.sum(onehot, axis=1, keepdims=True)
    gap = jnp.dot(onehot, hn, preferred_element_type=jnp.float32)
    gap = gap / jnp.maximum(cnt, 1.0)

    bcol = bc_ref[...]
    rows64 = lax.broadcasted_iota(jnp.int32, (B, 1), 0)

    def gsp_step(i, acc):
      m = jnp.max(jnp.where(bcol == i, hn, NEG), axis=0, keepdims=True)
      return jnp.where(rows64 == i, m, acc)

    gsp = lax.fori_loop(0, B, gsp_step, jnp.full((B, D), NEG, jnp.float32))
    gsp = jnp.where(cnt > 0.0, gsp, 0.0)

    gap = bn(gap, gapg_ref[...], gapb_ref[...])
    gsp = bn(gsp, gspg_ref[...], gspb_ref[...])
    out = jnp.concatenate([gap, gsp], axis=1)
    out = jnp.dot(out, h1w_ref[...],
                  preferred_element_type=jnp.float32) + h1b_ref[...]
    out = jnp.maximum(out, 0.0)
    out = bn(out, hn1g_ref[...], hn1b_ref[...])
    o_ref[...] = jnp.dot(out, lw_ref[...],
                         preferred_element_type=jnp.float32) + lb_ref[...]

  return pl.pallas_call(
      body,
      out_shape=jax.ShapeDtypeStruct((B, C), jnp.float32),
  )(hn_in, batch_r, batch_c, gap_g, gap_b, gsp_g, gsp_b,
    h1_w, h1_b, hn1_g, hn1_b, last_w, last_b)


def kernel(x, edge_index, batch, Wk1, Wq1, Wv1, Ws1, bk1, bq1, bv1, cb1,
           bn1_g, bn1_b, Wk2, Wq2, Wv2, Ws2, bk2, bq2, bv2, cb2, bn2_g,
           bn2_b, gap_g, gap_b, gsp_g, gsp_b, h1_W, h1_b, hn1_g, hn1_b,
           last_W, last_b):
  src = edge_index[0]
  dst = edge_index[1]
  zer = jnp.zeros((N, D), jnp.float32)

  w1 = jnp.concatenate([Wk1, Wq1, Wv1, Ws1], axis=1)
  b1 = jnp.concatenate([bk1, bq1, bv1, cb1]).reshape(1, 4 * D)
  w2 = jnp.concatenate([Wk2, Wq2, Wv2, Ws2], axis=1)
  b2 = jnp.concatenate([bk2, bq2, bv2, cb2]).reshape(1, 4 * D)

  kd1, qv1, xs1 = _proj(x, w1, b1)
  parts1 = _edge_agg(kd1, qv1, src, dst, zer)
  h1 = _relu_bn(parts1, xs1, bn1_g.reshape(1, D), bn1_b.reshape(1, D))
  kd2, qv2, xs2 = _proj(h1, w2, b2)
  parts2 = _edge_agg(kd2, qv2, src, dst, zer)
  h2 = _relu_bn(parts2, xs2, bn2_g.reshape(1, D), bn2_b.reshape(1, D))
  return _head(h2, batch.reshape(1, N), batch.reshape(N, 1),
               gap_g.reshape(1, D), gap_b.reshape(1, D),
               gsp_g.reshape(1, D), gsp_b.reshape(1, D),
               h1_W, h1_b.reshape(1, H), hn1_g.reshape(1, H),
               hn1_b.reshape(1, H), last_W, last_b.reshape(1, C))
